# baseline (speedup 1.0000x reference)
import jax
import jax.numpy as jnp
from jax import lax
from jax.experimental import pallas as pl
from jax.experimental.pallas import tpu as pltpu
from jax.experimental.pallas import tpu_sc as plsc

_NC = 2
_NS = 16
_NW = _NC * _NS
_C = 112
_L = 16
_NBUF = 3


def _sc_spmm(x, edata, wdata, zeros):
    n, d = x.shape
    npad = zeros.shape[0]
    tot = edata.shape[0]
    per_pair = tot // _NS
    nch_a = 3 * int(round(per_pair * 0.433 / 3))
    nch_b = per_pair - nch_a
    rows_per_tile = npad // _NS
    nvec = d // _L
    mesh = plsc.VectorSubcoreMesh(core_axis_name="c", subcore_axis_name="s",
                                  num_cores=_NC, num_subcores=_NS)

    def body(x_hbm, e_hbm, w_hbm, z_hbm, out_hbm,
             acc, ebuf, wbuf, rows0, rows1, rows2,
             gsem0, gsem1, gsem2, ssem0, ssem1, ssem2):
        cid = lax.axis_index("c")
        sid = lax.axis_index("s")
        base = jnp.where(cid == 0, sid * nch_a, _NS * nch_a + sid * nch_b)
        trip = jnp.where(cid == 0, nch_a, nch_b)
        rows = (rows0, rows1, rows2)
        gsem = (gsem0, gsem1, gsem2)
        ssem = (ssem0, ssem1, ssem2)

        r0 = sid * rows_per_tile
        pltpu.sync_copy(z_hbm.at[pl.ds(r0, rows_per_tile)],
                        acc.at[pl.ds(r0, rows_per_tile)])
        plsc.subcore_barrier()

        def fetch_and_gather(jj, s):
            pltpu.sync_copy(e_hbm.at[base + jj], ebuf.at[s])
            pltpu.sync_copy(w_hbm.at[base + jj], wbuf.at[s])
            pltpu.async_copy(x_hbm.at[ebuf.at[s, 0]], rows[s], gsem[s])

        fetch_and_gather(0, 0)
        fetch_and_gather(1, 1)

        def triple(t, carry):
            for b in range(_NBUF):
                jj = _NBUF * t + b
                sn = (b + 2) % _NBUF

                pltpu.make_async_copy(
                    x_hbm.at[ebuf.at[b, 0]], rows[b], gsem[b]).wait()

                @plsc.parallel_loop(0, _C // _L)
                def _(g):
                    wg = wbuf[b, pl.ds(g * _L, _L)]
                    for l in range(_L):
                        wvec = jnp.full((_L,), wg[l], dtype=jnp.float32)
                        e = g * _L + l
                        for k in range(nvec):
                            sl = pl.ds(k * _L, _L)
                            rows[b][e, sl] = rows[b][e, sl] * wvec

                @pl.when(jj >= 1)
                def _():
                    pltpu.make_async_copy(
                        rows[sn], acc.at[ebuf.at[sn, 1]], ssem[sn]).wait()

                @pl.when(jj + 2 < trip)
                def _():
                    fetch_and_gather(jj + 2, sn)

                pltpu.make_async_copy(
                    rows[b], acc.at[ebuf.at[b, 1]], ssem[b]).start(add=True)
            return carry

        lax.fori_loop(0, trip // _NBUF, triple, 0)

        pltpu.make_async_copy(
            rows[2], acc.at[ebuf.at[2, 1]], ssem[2]).wait()

        plsc.subcore_barrier()
        pltpu.sync_copy(acc.at[pl.ds(r0, rows_per_tile)],
                        out_hbm.at[cid, pl.ds(r0, rows_per_tile)])

    return pl.kernel(
        body,
        out_type=jax.ShapeDtypeStruct((_NC, npad, d), jnp.float32),
        mesh=mesh,
        scratch_types=[
            pltpu.VMEM_SHARED((npad, d), jnp.float32),
            pltpu.VMEM((_NBUF, 2, _C), jnp.int32),
            pltpu.VMEM((_NBUF, _C), jnp.float32),
            pltpu.VMEM((_C, d), jnp.float32),
            pltpu.VMEM((_C, d), jnp.float32),
            pltpu.VMEM((_C, d), jnp.float32),
            pltpu.SemaphoreType.DMA,
            pltpu.SemaphoreType.DMA,
            pltpu.SemaphoreType.DMA,
            pltpu.SemaphoreType.DMA,
            pltpu.SemaphoreType.DMA,
            pltpu.SemaphoreType.DMA,
        ],
    )(x, edata, wdata, zeros)


def _tc_linear(partial, W, b2, n):
    d = partial.shape[2]
    blk = 1000

    def body(p_ref, w_ref, b_ref, o_ref):
        h = p_ref[0] + p_ref[1]
        o_ref[...] = lax.dot_general(
            h, w_ref[...], (((1,), (1,)), ((), ())),
            preferred_element_type=jnp.float32) + b_ref[...]

    return pl.pallas_call(
        body,
        grid=(n // blk,),
        in_specs=[
            pl.BlockSpec((2, blk, d), lambda i: (0, i, 0)),
            pl.BlockSpec((d, d), lambda i: (0, 0)),
            pl.BlockSpec((1, d), lambda i: (0, 0)),
        ],
        out_specs=pl.BlockSpec((blk, d), lambda i: (i, 0)),
        out_shape=jax.ShapeDtypeStruct((n, d), jnp.float32),
    )(partial, W, b2)


def kernel(x, edge_index, edge_weight, W, b):
    n, d = x.shape
    e = edge_index.shape[1]
    quantum = _NW * _C * _NBUF
    ep = quantum * ((e + quantum - 1) // quantum)
    pad = ep - e
    nch = ep // (_NW * _C)
    tot = ep // _C
    src = jnp.concatenate(
        [edge_index[0].astype(jnp.int32), jnp.zeros((pad,), jnp.int32)])
    dst = jnp.concatenate(
        [edge_index[1].astype(jnp.int32), jnp.zeros((pad,), jnp.int32)])
    w = jnp.concatenate(
        [edge_weight.astype(jnp.float32), jnp.zeros((pad,), jnp.float32)])
    edata = jnp.stack([src.reshape(tot, _C), dst.reshape(tot, _C)], axis=1)
    wdata = w.reshape(tot, _C)
    nq = 8 * _NS
    npad = nq * ((n + nq - 1) // nq)
    zeros = jnp.zeros((npad, d), jnp.float32)
    partial = _sc_spmm(x, edata, wdata, zeros)
    return _tc_linear(partial, W, b.reshape(1, d), n)


# speedup vs baseline: 1.0738x; 1.0738x over previous
import jax
import jax.numpy as jnp
from jax import lax
from jax.experimental import pallas as pl
from jax.experimental.pallas import tpu as pltpu
from jax.experimental.pallas import tpu_sc as plsc

_NC = 2
_NS = 16
_NW = _NC * _NS
_C = 112
_L = 16
_NBUF = 3


def _sc_spmm(x, edata, wdata, zeros):
    n, d = x.shape
    npad = zeros.shape[0]
    tot = edata.shape[0]
    per_pair = tot // _NS
    nch_a = 3 * int(round(per_pair * 0.567 / 3))
    nch_b = per_pair - nch_a
    rows_per_tile = npad // _NS
    nvec = d // _L
    mesh = plsc.VectorSubcoreMesh(core_axis_name="c", subcore_axis_name="s",
                                  num_cores=_NC, num_subcores=_NS)

    def body(x_hbm, e_hbm, w_hbm, z_hbm, out_hbm,
             acc, ebuf, wbuf, rows0, rows1, rows2,
             gsem0, gsem1, gsem2, ssem0, ssem1, ssem2):
        cid = lax.axis_index("c")
        sid = lax.axis_index("s")
        base = jnp.where(cid == 0, sid * nch_a, _NS * nch_a + sid * nch_b)
        trip = jnp.where(cid == 0, nch_a, nch_b)
        rows = (rows0, rows1, rows2)
        gsem = (gsem0, gsem1, gsem2)
        ssem = (ssem0, ssem1, ssem2)

        r0 = sid * rows_per_tile
        pltpu.sync_copy(z_hbm.at[pl.ds(r0, rows_per_tile)],
                        acc.at[pl.ds(r0, rows_per_tile)])
        plsc.subcore_barrier()

        def fetch_and_gather(jj, s):
            pltpu.sync_copy(e_hbm.at[base + jj], ebuf.at[s])
            pltpu.sync_copy(w_hbm.at[base + jj], wbuf.at[s])
            pltpu.async_copy(x_hbm.at[ebuf.at[s, 0]], rows[s], gsem[s])

        fetch_and_gather(0, 0)
        fetch_and_gather(1, 1)

        def triple(t, carry):
            for b in range(_NBUF):
                jj = _NBUF * t + b
                sn = (b + 2) % _NBUF

                pltpu.make_async_copy(
                    x_hbm.at[ebuf.at[b, 0]], rows[b], gsem[b]).wait()

                @plsc.parallel_loop(0, _C // _L)
                def _(g):
                    wg = wbuf[b, pl.ds(g * _L, _L)]
                    for l in range(_L):
                        wvec = jnp.full((_L,), wg[l], dtype=jnp.float32)
                        e = g * _L + l
                        for k in range(nvec):
                            sl = pl.ds(k * _L, _L)
                            rows[b][e, sl] = rows[b][e, sl] * wvec

                @pl.when(jj >= 1)
                def _():
                    pltpu.make_async_copy(
                        rows[sn], acc.at[ebuf.at[sn, 1]], ssem[sn]).wait()

                @pl.when(jj + 2 < trip)
                def _():
                    fetch_and_gather(jj + 2, sn)

                pltpu.make_async_copy(
                    rows[b], acc.at[ebuf.at[b, 1]], ssem[b]).start(add=True)
            return carry

        lax.fori_loop(0, trip // _NBUF, triple, 0)

        pltpu.make_async_copy(
            rows[2], acc.at[ebuf.at[2, 1]], ssem[2]).wait()

        plsc.subcore_barrier()
        pltpu.sync_copy(acc.at[pl.ds(r0, rows_per_tile)],
                        out_hbm.at[cid, pl.ds(r0, rows_per_tile)])

    return pl.kernel(
        body,
        out_type=jax.ShapeDtypeStruct((_NC, npad, d), jnp.float32),
        mesh=mesh,
        scratch_types=[
            pltpu.VMEM_SHARED((npad, d), jnp.float32),
            pltpu.VMEM((_NBUF, 2, _C), jnp.int32),
            pltpu.VMEM((_NBUF, _C), jnp.float32),
            pltpu.VMEM((_C, d), jnp.float32),
            pltpu.VMEM((_C, d), jnp.float32),
            pltpu.VMEM((_C, d), jnp.float32),
            pltpu.SemaphoreType.DMA,
            pltpu.SemaphoreType.DMA,
            pltpu.SemaphoreType.DMA,
            pltpu.SemaphoreType.DMA,
            pltpu.SemaphoreType.DMA,
            pltpu.SemaphoreType.DMA,
        ],
    )(x, edata, wdata, zeros)


def _tc_linear(partial, W, b2, n):
    d = partial.shape[2]
    blk = 1000

    def body(p_ref, w_ref, b_ref, o_ref):
        h = p_ref[0] + p_ref[1]
        o_ref[...] = lax.dot_general(
            h, w_ref[...], (((1,), (1,)), ((), ())),
            preferred_element_type=jnp.float32) + b_ref[...]

    return pl.pallas_call(
        body,
        grid=(n // blk,),
        in_specs=[
            pl.BlockSpec((2, blk, d), lambda i: (0, i, 0)),
            pl.BlockSpec((d, d), lambda i: (0, 0)),
            pl.BlockSpec((1, d), lambda i: (0, 0)),
        ],
        out_specs=pl.BlockSpec((blk, d), lambda i: (i, 0)),
        out_shape=jax.ShapeDtypeStruct((n, d), jnp.float32),
    )(partial, W, b2)


def kernel(x, edge_index, edge_weight, W, b):
    n, d = x.shape
    e = edge_index.shape[1]
    quantum = _NW * _C * _NBUF
    ep = quantum * ((e + quantum - 1) // quantum)
    pad = ep - e
    nch = ep // (_NW * _C)
    tot = ep // _C
    src = jnp.concatenate(
        [edge_index[0].astype(jnp.int32), jnp.zeros((pad,), jnp.int32)])
    dst = jnp.concatenate(
        [edge_index[1].astype(jnp.int32), jnp.zeros((pad,), jnp.int32)])
    w = jnp.concatenate(
        [edge_weight.astype(jnp.float32), jnp.zeros((pad,), jnp.float32)])
    edata = jnp.stack([src.reshape(tot, _C), dst.reshape(tot, _C)], axis=1)
    wdata = w.reshape(tot, _C)
    nq = 8 * _NS
    npad = nq * ((n + nq - 1) // nq)
    zeros = jnp.zeros((npad, d), jnp.float32)
    partial = _sc_spmm(x, edata, wdata, zeros)
    return _tc_linear(partial, W, b.reshape(1, d), n)


# speedup vs baseline: 1.0923x; 1.0172x over previous
import jax
import jax.numpy as jnp
from jax import lax
from jax.experimental import pallas as pl
from jax.experimental.pallas import tpu as pltpu
from jax.experimental.pallas import tpu_sc as plsc

_NC = 2
_NS = 16
_NW = _NC * _NS
_C = 112
_L = 16
_NBUF = 3


def _sc_spmm(x, edata, wdata, zeros):
    n, d = x.shape
    npad = zeros.shape[0]
    tot = edata.shape[0]
    per_pair = tot // _NS
    nch_a = 3 * int(round(per_pair * 0.583 / 3))
    nch_b = per_pair - nch_a
    rows_per_tile = npad // _NS
    nvec = d // _L
    mesh = plsc.VectorSubcoreMesh(core_axis_name="c", subcore_axis_name="s",
                                  num_cores=_NC, num_subcores=_NS)

    def body(x_hbm, e_hbm, w_hbm, z_hbm, out_hbm,
             acc, ebuf, wbuf, rows0, rows1, rows2,
             gsem0, gsem1, gsem2, ssem0, ssem1, ssem2):
        cid = lax.axis_index("c")
        sid = lax.axis_index("s")
        base = jnp.where(cid == 0, sid * nch_a, _NS * nch_a + sid * nch_b)
        trip = jnp.where(cid == 0, nch_a, nch_b)
        rows = (rows0, rows1, rows2)
        gsem = (gsem0, gsem1, gsem2)
        ssem = (ssem0, ssem1, ssem2)

        r0 = sid * rows_per_tile
        pltpu.sync_copy(z_hbm.at[pl.ds(r0, rows_per_tile)],
                        acc.at[pl.ds(r0, rows_per_tile)])
        plsc.subcore_barrier()

        def fetch_and_gather(jj, s):
            pltpu.sync_copy(e_hbm.at[base + jj], ebuf.at[s])
            pltpu.sync_copy(w_hbm.at[base + jj], wbuf.at[s])
            pltpu.async_copy(x_hbm.at[ebuf.at[s, 0]], rows[s], gsem[s])

        fetch_and_gather(0, 0)
        fetch_and_gather(1, 1)

        def triple(t, carry):
            for b in range(_NBUF):
                jj = _NBUF * t + b
                sn = (b + 2) % _NBUF

                pltpu.make_async_copy(
                    x_hbm.at[ebuf.at[b, 0]], rows[b], gsem[b]).wait()

                @plsc.parallel_loop(0, _C // _L)
                def _(g):
                    wg = wbuf[b, pl.ds(g * _L, _L)]
                    for l in range(_L):
                        wvec = jnp.full((_L,), wg[l], dtype=jnp.float32)
                        e = g * _L + l
                        for k in range(nvec):
                            sl = pl.ds(k * _L, _L)
                            rows[b][e, sl] = rows[b][e, sl] * wvec

                @pl.when(jj >= 1)
                def _():
                    pltpu.make_async_copy(
                        rows[sn], acc.at[ebuf.at[sn, 1]], ssem[sn]).wait()

                @pl.when(jj + 2 < trip)
                def _():
                    fetch_and_gather(jj + 2, sn)

                pltpu.make_async_copy(
                    rows[b], acc.at[ebuf.at[b, 1]], ssem[b]).start(add=True)
            return carry

        lax.fori_loop(0, trip // _NBUF, triple, 0)

        pltpu.make_async_copy(
            rows[2], acc.at[ebuf.at[2, 1]], ssem[2]).wait()

        plsc.subcore_barrier()
        pltpu.sync_copy(acc.at[pl.ds(r0, rows_per_tile)],
                        out_hbm.at[cid, pl.ds(r0, rows_per_tile)])

    return pl.kernel(
        body,
        out_type=jax.ShapeDtypeStruct((_NC, npad, d), jnp.float32),
        mesh=mesh,
        scratch_types=[
            pltpu.VMEM_SHARED((npad, d), jnp.float32),
            pltpu.VMEM((_NBUF, 2, _C), jnp.int32),
            pltpu.VMEM((_NBUF, _C), jnp.float32),
            pltpu.VMEM((_C, d), jnp.float32),
            pltpu.VMEM((_C, d), jnp.float32),
            pltpu.VMEM((_C, d), jnp.float32),
            pltpu.SemaphoreType.DMA,
            pltpu.SemaphoreType.DMA,
            pltpu.SemaphoreType.DMA,
            pltpu.SemaphoreType.DMA,
            pltpu.SemaphoreType.DMA,
            pltpu.SemaphoreType.DMA,
        ],
    )(x, edata, wdata, zeros)


def _tc_linear(partial, W, b2, n):
    d = partial.shape[2]
    blk = 1000

    def body(p_ref, w_ref, b_ref, o_ref):
        h = p_ref[0] + p_ref[1]
        o_ref[...] = lax.dot_general(
            h, w_ref[...], (((1,), (1,)), ((), ())),
            preferred_element_type=jnp.float32) + b_ref[...]

    return pl.pallas_call(
        body,
        grid=(n // blk,),
        in_specs=[
            pl.BlockSpec((2, blk, d), lambda i: (0, i, 0)),
            pl.BlockSpec((d, d), lambda i: (0, 0)),
            pl.BlockSpec((1, d), lambda i: (0, 0)),
        ],
        out_specs=pl.BlockSpec((blk, d), lambda i: (i, 0)),
        out_shape=jax.ShapeDtypeStruct((n, d), jnp.float32),
    )(partial, W, b2)


def kernel(x, edge_index, edge_weight, W, b):
    n, d = x.shape
    e = edge_index.shape[1]
    quantum = _NW * _C * _NBUF
    ep = quantum * ((e + quantum - 1) // quantum)
    pad = ep - e
    nch = ep // (_NW * _C)
    tot = ep // _C
    src = jnp.concatenate(
        [edge_index[0].astype(jnp.int32), jnp.zeros((pad,), jnp.int32)])
    dst = jnp.concatenate(
        [edge_index[1].astype(jnp.int32), jnp.zeros((pad,), jnp.int32)])
    w = jnp.concatenate(
        [edge_weight.astype(jnp.float32), jnp.zeros((pad,), jnp.float32)])
    edata = jnp.stack([src.reshape(tot, _C), dst.reshape(tot, _C)], axis=1)
    wdata = w.reshape(tot, _C)
    nq = 8 * _NS
    npad = nq * ((n + nq - 1) // nq)
    zeros = jnp.zeros((npad, d), jnp.float32)
    partial = _sc_spmm(x, edata, wdata, zeros)
    return _tc_linear(partial, W, b.reshape(1, d), n)


# speedup vs baseline: 1.1014x; 1.0083x over previous
import jax
import jax.numpy as jnp
from jax import lax
from jax.experimental import pallas as pl
from jax.experimental.pallas import tpu as pltpu
from jax.experimental.pallas import tpu_sc as plsc

_NC = 2
_NS = 16
_NW = _NC * _NS
_C = 112
_L = 16
_NBUF = 3


def _sc_spmm(x, edata, wdata, zeros):
    n, d = x.shape
    npad = zeros.shape[0]
    tot = edata.shape[0]
    per_pair = tot // _NS
    nch_a = 3 * int(round(per_pair * 0.600 / 3))
    nch_b = per_pair - nch_a
    rows_per_tile = npad // _NS
    nvec = d // _L
    mesh = plsc.VectorSubcoreMesh(core_axis_name="c", subcore_axis_name="s",
                                  num_cores=_NC, num_subcores=_NS)

    def body(x_hbm, e_hbm, w_hbm, z_hbm, out_hbm,
             acc, ebuf, wbuf, rows0, rows1, rows2,
             gsem0, gsem1, gsem2, ssem0, ssem1, ssem2):
        cid = lax.axis_index("c")
        sid = lax.axis_index("s")
        base = jnp.where(cid == 0, sid * nch_a, _NS * nch_a + sid * nch_b)
        trip = jnp.where(cid == 0, nch_a, nch_b)
        rows = (rows0, rows1, rows2)
        gsem = (gsem0, gsem1, gsem2)
        ssem = (ssem0, ssem1, ssem2)

        r0 = sid * rows_per_tile
        pltpu.sync_copy(z_hbm.at[pl.ds(r0, rows_per_tile)],
                        acc.at[pl.ds(r0, rows_per_tile)])
        plsc.subcore_barrier()

        def fetch_and_gather(jj, s):
            pltpu.sync_copy(e_hbm.at[base + jj], ebuf.at[s])
            pltpu.sync_copy(w_hbm.at[base + jj], wbuf.at[s])
            pltpu.async_copy(x_hbm.at[ebuf.at[s, 0]], rows[s], gsem[s])

        fetch_and_gather(0, 0)
        fetch_and_gather(1, 1)

        def triple(t, carry):
            for b in range(_NBUF):
                jj = _NBUF * t + b
                sn = (b + 2) % _NBUF

                pltpu.make_async_copy(
                    x_hbm.at[ebuf.at[b, 0]], rows[b], gsem[b]).wait()

                @plsc.parallel_loop(0, _C // _L)
                def _(g):
                    wg = wbuf[b, pl.ds(g * _L, _L)]
                    for l in range(_L):
                        wvec = jnp.full((_L,), wg[l], dtype=jnp.float32)
                        e = g * _L + l
                        for k in range(nvec):
                            sl = pl.ds(k * _L, _L)
                            rows[b][e, sl] = rows[b][e, sl] * wvec

                @pl.when(jj >= 1)
                def _():
                    pltpu.make_async_copy(
                        rows[sn], acc.at[ebuf.at[sn, 1]], ssem[sn]).wait()

                @pl.when(jj + 2 < trip)
                def _():
                    fetch_and_gather(jj + 2, sn)

                pltpu.make_async_copy(
                    rows[b], acc.at[ebuf.at[b, 1]], ssem[b]).start(add=True)
            return carry

        lax.fori_loop(0, trip // _NBUF, triple, 0)

        pltpu.make_async_copy(
            rows[2], acc.at[ebuf.at[2, 1]], ssem[2]).wait()

        plsc.subcore_barrier()
        pltpu.sync_copy(acc.at[pl.ds(r0, rows_per_tile)],
                        out_hbm.at[cid, pl.ds(r0, rows_per_tile)])

    return pl.kernel(
        body,
        out_type=jax.ShapeDtypeStruct((_NC, npad, d), jnp.float32),
        mesh=mesh,
        scratch_types=[
            pltpu.VMEM_SHARED((npad, d), jnp.float32),
            pltpu.VMEM((_NBUF, 2, _C), jnp.int32),
            pltpu.VMEM((_NBUF, _C), jnp.float32),
            pltpu.VMEM((_C, d), jnp.float32),
            pltpu.VMEM((_C, d), jnp.float32),
            pltpu.VMEM((_C, d), jnp.float32),
            pltpu.SemaphoreType.DMA,
            pltpu.SemaphoreType.DMA,
            pltpu.SemaphoreType.DMA,
            pltpu.SemaphoreType.DMA,
            pltpu.SemaphoreType.DMA,
            pltpu.SemaphoreType.DMA,
        ],
    )(x, edata, wdata, zeros)


def _tc_linear(partial, W, b2, n):
    d = partial.shape[2]
    blk = 1000

    def body(p_ref, w_ref, b_ref, o_ref):
        h = p_ref[0] + p_ref[1]
        o_ref[...] = lax.dot_general(
            h, w_ref[...], (((1,), (1,)), ((), ())),
            preferred_element_type=jnp.float32) + b_ref[...]

    return pl.pallas_call(
        body,
        grid=(n // blk,),
        in_specs=[
            pl.BlockSpec((2, blk, d), lambda i: (0, i, 0)),
            pl.BlockSpec((d, d), lambda i: (0, 0)),
            pl.BlockSpec((1, d), lambda i: (0, 0)),
        ],
        out_specs=pl.BlockSpec((blk, d), lambda i: (i, 0)),
        out_shape=jax.ShapeDtypeStruct((n, d), jnp.float32),
    )(partial, W, b2)


def kernel(x, edge_index, edge_weight, W, b):
    n, d = x.shape
    e = edge_index.shape[1]
    quantum = _NW * _C * _NBUF
    ep = quantum * ((e + quantum - 1) // quantum)
    pad = ep - e
    nch = ep // (_NW * _C)
    tot = ep // _C
    src = jnp.concatenate(
        [edge_index[0].astype(jnp.int32), jnp.zeros((pad,), jnp.int32)])
    dst = jnp.concatenate(
        [edge_index[1].astype(jnp.int32), jnp.zeros((pad,), jnp.int32)])
    w = jnp.concatenate(
        [edge_weight.astype(jnp.float32), jnp.zeros((pad,), jnp.float32)])
    edata = jnp.stack([src.reshape(tot, _C), dst.reshape(tot, _C)], axis=1)
    wdata = w.reshape(tot, _C)
    nq = 8 * _NS
    npad = nq * ((n + nq - 1) // nq)
    zeros = jnp.zeros((npad, d), jnp.float32)
    partial = _sc_spmm(x, edata, wdata, zeros)
    return _tc_linear(partial, W, b.reshape(1, d), n)


# speedup vs baseline: 1.1048x; 1.0031x over previous
import jax
import jax.numpy as jnp
from jax import lax
from jax.experimental import pallas as pl
from jax.experimental.pallas import tpu as pltpu
from jax.experimental.pallas import tpu_sc as plsc

_NC = 2
_NS = 16
_NW = _NC * _NS
_C = 112
_L = 16
_NBUF = 3


def _sc_spmm(x, edata, wdata, zeros):
    n, d = x.shape
    npad = zeros.shape[0]
    tot = edata.shape[0]
    per_pair = tot // _NS
    nch_a = 3 * int(round(per_pair * 0.617 / 3))
    nch_b = per_pair - nch_a
    rows_per_tile = npad // _NS
    nvec = d // _L
    mesh = plsc.VectorSubcoreMesh(core_axis_name="c", subcore_axis_name="s",
                                  num_cores=_NC, num_subcores=_NS)

    def body(x_hbm, e_hbm, w_hbm, z_hbm, out_hbm,
             acc, ebuf, wbuf, rows0, rows1, rows2,
             gsem0, gsem1, gsem2, ssem0, ssem1, ssem2):
        cid = lax.axis_index("c")
        sid = lax.axis_index("s")
        base = jnp.where(cid == 0, sid * nch_a, _NS * nch_a + sid * nch_b)
        trip = jnp.where(cid == 0, nch_a, nch_b)
        rows = (rows0, rows1, rows2)
        gsem = (gsem0, gsem1, gsem2)
        ssem = (ssem0, ssem1, ssem2)

        r0 = sid * rows_per_tile
        pltpu.sync_copy(z_hbm.at[pl.ds(r0, rows_per_tile)],
                        acc.at[pl.ds(r0, rows_per_tile)])
        plsc.subcore_barrier()

        def fetch_and_gather(jj, s):
            pltpu.sync_copy(e_hbm.at[base + jj], ebuf.at[s])
            pltpu.sync_copy(w_hbm.at[base + jj], wbuf.at[s])
            pltpu.async_copy(x_hbm.at[ebuf.at[s, 0]], rows[s], gsem[s])

        fetch_and_gather(0, 0)
        fetch_and_gather(1, 1)

        def triple(t, carry):
            for b in range(_NBUF):
                jj = _NBUF * t + b
                sn = (b + 2) % _NBUF

                pltpu.make_async_copy(
                    x_hbm.at[ebuf.at[b, 0]], rows[b], gsem[b]).wait()

                @plsc.parallel_loop(0, _C // _L)
                def _(g):
                    wg = wbuf[b, pl.ds(g * _L, _L)]
                    for l in range(_L):
                        wvec = jnp.full((_L,), wg[l], dtype=jnp.float32)
                        e = g * _L + l
                        for k in range(nvec):
                            sl = pl.ds(k * _L, _L)
                            rows[b][e, sl] = rows[b][e, sl] * wvec

                @pl.when(jj >= 1)
                def _():
                    pltpu.make_async_copy(
                        rows[sn], acc.at[ebuf.at[sn, 1]], ssem[sn]).wait()

                @pl.when(jj + 2 < trip)
                def _():
                    fetch_and_gather(jj + 2, sn)

                pltpu.make_async_copy(
                    rows[b], acc.at[ebuf.at[b, 1]], ssem[b]).start(add=True)
            return carry

        lax.fori_loop(0, trip // _NBUF, triple, 0)

        pltpu.make_async_copy(
            rows[2], acc.at[ebuf.at[2, 1]], ssem[2]).wait()

        plsc.subcore_barrier()
        pltpu.sync_copy(acc.at[pl.ds(r0, rows_per_tile)],
                        out_hbm.at[cid, pl.ds(r0, rows_per_tile)])

    return pl.kernel(
        body,
        out_type=jax.ShapeDtypeStruct((_NC, npad, d), jnp.float32),
        mesh=mesh,
        scratch_types=[
            pltpu.VMEM_SHARED((npad, d), jnp.float32),
            pltpu.VMEM((_NBUF, 2, _C), jnp.int32),
            pltpu.VMEM((_NBUF, _C), jnp.float32),
            pltpu.VMEM((_C, d), jnp.float32),
            pltpu.VMEM((_C, d), jnp.float32),
            pltpu.VMEM((_C, d), jnp.float32),
            pltpu.SemaphoreType.DMA,
            pltpu.SemaphoreType.DMA,
            pltpu.SemaphoreType.DMA,
            pltpu.SemaphoreType.DMA,
            pltpu.SemaphoreType.DMA,
            pltpu.SemaphoreType.DMA,
        ],
    )(x, edata, wdata, zeros)


def _tc_linear(partial, W, b2, n):
    d = partial.shape[2]
    blk = 1000

    def body(p_ref, w_ref, b_ref, o_ref):
        h = p_ref[0] + p_ref[1]
        o_ref[...] = lax.dot_general(
            h, w_ref[...], (((1,), (1,)), ((), ())),
            preferred_element_type=jnp.float32) + b_ref[...]

    return pl.pallas_call(
        body,
        grid=(n // blk,),
        in_specs=[
            pl.BlockSpec((2, blk, d), lambda i: (0, i, 0)),
            pl.BlockSpec((d, d), lambda i: (0, 0)),
            pl.BlockSpec((1, d), lambda i: (0, 0)),
        ],
        out_specs=pl.BlockSpec((blk, d), lambda i: (i, 0)),
        out_shape=jax.ShapeDtypeStruct((n, d), jnp.float32),
    )(partial, W, b2)


def kernel(x, edge_index, edge_weight, W, b):
    n, d = x.shape
    e = edge_index.shape[1]
    quantum = _NW * _C * _NBUF
    ep = quantum * ((e + quantum - 1) // quantum)
    pad = ep - e
    nch = ep // (_NW * _C)
    tot = ep // _C
    src = jnp.concatenate(
        [edge_index[0].astype(jnp.int32), jnp.zeros((pad,), jnp.int32)])
    dst = jnp.concatenate(
        [edge_index[1].astype(jnp.int32), jnp.zeros((pad,), jnp.int32)])
    w = jnp.concatenate(
        [edge_weight.astype(jnp.float32), jnp.zeros((pad,), jnp.float32)])
    edata = jnp.stack([src.reshape(tot, _C), dst.reshape(tot, _C)], axis=1)
    wdata = w.reshape(tot, _C)
    nq = 8 * _NS
    npad = nq * ((n + nq - 1) // nq)
    zeros = jnp.zeros((npad, d), jnp.float32)
    partial = _sc_spmm(x, edata, wdata, zeros)
    return _tc_linear(partial, W, b.reshape(1, d), n)
